# baseline (speedup 1.0000x reference)
import functools

import jax
import jax.numpy as jnp
from jax import lax
from jax.experimental import pallas as pl
from jax.experimental.pallas import tpu as pltpu
from jax.experimental.pallas import tpu_sc as plsc

_N = 8
_F = 8
_H = 32
_BUDGET_SECONDS = 60 * 60.0
_GUARD = 5.0

_O_SCAL = 0
_O_H = 16
_O_W1T = 80
_O_W2T = 336
_O_WIHT = 592
_O_WHHT = 1616
_O_BIH = 1872
_O_BHH = 1904
_O_MSGB = 1936
_O_DECW = 1968
_P_LEN = 1984


def _sigmoid(x):
    return 1.0 / (1.0 + jnp.exp(-x))


def _tanh(x):
    return 2.0 / (1.0 + jnp.exp(-2.0 * x)) - 1.0


def _sc_body(params_hbm, out_hbm, p, tmp):
    @pl.when((lax.axis_index("c") == 0) & (lax.axis_index("s") == 0))
    def _():
        pltpu.sync_copy(params_hbm, p)

        iota = lax.iota(jnp.int32, 16)
        zeros16 = jnp.zeros((16,), jnp.float32)
        mask_lo = iota < 8
        tmp[pl.ds(32, 16)] = zeros16

        scal = p[pl.ds(_O_SCAL, 16)]
        ent = scal[0]
        rt = scal[1]
        decb = scal[2]
        rem_norm = jnp.maximum(_BUDGET_SECONDS - rt, 0.0) * (1.0 / _BUDGET_SECONDS)
        featv = jnp.where(iota == 0, ent, jnp.where(iota == 1, rem_norm, 0.0))

        hv = []
        for d in range(_N):
            raw = p[pl.ds(_O_H + 8 * d, 16)]
            hv.append(jnp.where(mask_lo, raw + featv, 0.0))
        hsum = hv[0]
        for d in range(1, _N):
            hsum = hsum + hv[d]

        q0 = _N * p[pl.ds(_O_MSGB, 16)]
        q1 = _N * p[pl.ds(_O_MSGB + 16, 16)]
        for f in range(_F):
            s = hsum[f]
            q0 = q0 + s * p[pl.ds(_O_W1T + 32 * f, 16)]
            q1 = q1 + s * p[pl.ds(_O_W1T + 32 * f + 16, 16)]
        agg0 = []
        agg1 = []
        for d in range(_N):
            a0, a1 = q0, q1
            for f in range(_F):
                s8 = 8.0 * hv[d][f]
                a0 = a0 + s8 * p[pl.ds(_O_W2T + 32 * f, 16)]
                a1 = a1 + s8 * p[pl.ds(_O_W2T + 32 * f + 16, 16)]
            agg0.append(a0)
            agg1.append(a1)

        gi0 = [p[pl.ds(_O_BIH, 16)]] * _N
        gi1 = [p[pl.ds(_O_BIH + 16, 16)]] * _N
        for j in range(_H):
            w0 = p[pl.ds(_O_WIHT + 32 * j, 16)]
            w1 = p[pl.ds(_O_WIHT + 32 * j + 16, 16)]
            for d in range(_N):
                s = agg0[d][j] if j < 16 else agg1[d][j - 16]
                gi0[d] = gi0[d] + s * w0
                gi1[d] = gi1[d] + s * w1
        gh0 = [p[pl.ds(_O_BHH, 16)]] * _N
        gh1 = [p[pl.ds(_O_BHH + 16, 16)]] * _N
        for f in range(_F):
            w0 = p[pl.ds(_O_WHHT + 32 * f, 16)]
            w1 = p[pl.ds(_O_WHHT + 32 * f + 16, 16)]
            for d in range(_N):
                s = hv[d][f]
                gh0[d] = gh0[d] + s * w0
                gh1[d] = gh1[d] + s * w1

        hm = zeros16
        for d in range(_N):
            s_rz = _sigmoid(gi0[d] + gh0[d])
            n_d = _tanh(gi1[d] + s_rz * gh1[d])
            tmp[pl.ds(16, 16)] = s_rz
            zv = tmp[pl.ds(24, 16)]
            hm = hm + (1.0 - zv) * n_d + zv * hv[d]
        hm = hm * (1.0 / _N)

        decv = p[pl.ds(_O_DECW, 16)]
        prod = hm * decv
        lr_log = decb
        for f in range(_F):
            lr_log = lr_log + prod[f]
        lrv = jnp.exp(jnp.broadcast_to(lr_log, (16,)))
        lrv = jnp.minimum(jnp.maximum(lrv, 0.001), _GUARD)
        tmp[pl.ds(48, 16)] = lrv
        pltpu.sync_copy(tmp.at[pl.ds(48, 16)], out_hbm)


def _sc_call(packed):
    mesh = plsc.VectorSubcoreMesh(
        core_axis_name="c", subcore_axis_name="s", num_cores=2, num_subcores=16
    )
    return pl.kernel(
        _sc_body,
        out_type=jax.ShapeDtypeStruct((16,), jnp.float32),
        mesh=mesh,
        scratch_types=[
            pltpu.VMEM((_P_LEN,), jnp.float32),
            pltpu.VMEM((64,), jnp.float32),
        ],
    )(packed)


def kernel(entropy, h_param, running_time, msg_W, msg_b, gru_Wih, gru_Whh,
           gru_bih, gru_bhh, dec_W, dec_b):
    f32 = jnp.float32
    zeros8 = jnp.zeros((8,), f32)
    packed = jnp.concatenate([
        jnp.float32(entropy)[None], running_time.astype(f32),
        dec_b.astype(f32), jnp.zeros((13,), f32),
        h_param.reshape(-1).astype(f32),
        msg_W[:, :_F].T.reshape(-1).astype(f32),
        msg_W[:, _F:].T.reshape(-1).astype(f32),
        jnp.concatenate([gru_Wih.astype(f32), jnp.zeros((8, _H), f32)], axis=0).T.reshape(-1),
        jnp.concatenate([gru_Whh.astype(f32), jnp.zeros((8, _F), f32)], axis=0).T.reshape(-1),
        gru_bih.astype(f32), zeros8,
        gru_bhh.astype(f32), zeros8,
        msg_b.astype(f32),
        dec_W.reshape(-1).astype(f32), zeros8,
    ])
    out = _sc_call(packed)
    return out[:1]


# speedup vs baseline: 1.0906x; 1.0906x over previous
import jax
import jax.numpy as jnp
from jax import lax
from jax.experimental import pallas as pl
from jax.experimental.pallas import tpu as pltpu
from jax.experimental.pallas import tpu_sc as plsc

_N = 8
_F = 8
_H = 32
_BUDGET_SECONDS = 60 * 60.0
_GUARD = 5.0

_O_SCAL = 0
_O_H = 16
_O_W1T = 80
_O_W2T = 336
_O_WIHT = 592
_O_WHHT = 1616
_O_BIH = 1872
_O_BHH = 1904
_O_MSGB = 1936
_O_DECW = 1968
_P_LEN = 1984


def _bcast(v, j):
    idx = jnp.full((16, 1), j, jnp.int32)
    dnums = lax.GatherDimensionNumbers(
        offset_dims=(), collapsed_slice_dims=(0,), start_index_map=(0,))
    return lax.gather(v, idx, dnums, (1,),
                      mode=lax.GatherScatterMode.PROMISE_IN_BOUNDS)


def _sigmoid(x):
    return 1.0 / (1.0 + jnp.exp(-x))


def _tanh(x):
    return 2.0 / (1.0 + jnp.exp(-2.0 * x)) - 1.0


def _sc_body(params_hbm, out_hbm, p, hbuf, tmp):
    @pl.when((lax.axis_index("c") == 0) & (lax.axis_index("s") == 0))
    def _():
        pltpu.sync_copy(params_hbm, p)

        iota = lax.iota(jnp.int32, 16)
        zeros16 = jnp.zeros((16,), jnp.float32)
        mask_lo = iota < 8
        tmp[pl.ds(32, 16)] = zeros16

        scal = p[pl.ds(_O_SCAL, 16)]
        ent = scal[0]
        rt = scal[1]
        decb = scal[2]
        rem_norm = jnp.maximum(_BUDGET_SECONDS - rt, 0.0) * (1.0 / _BUDGET_SECONDS)
        featv = jnp.where(iota == 0, ent, jnp.where(iota == 1, rem_norm, 0.0))

        hsum = zeros16
        for d in range(_N):
            raw = p[pl.ds(_O_H + 8 * d, 16)]
            hval = jnp.where(mask_lo, raw + featv, 0.0)
            hbuf[pl.ds(16 * d, 16)] = hval
            hsum = hsum + hval

        q0 = _N * p[pl.ds(_O_MSGB, 16)]
        q1 = _N * p[pl.ds(_O_MSGB + 16, 16)]
        for f in range(_F):
            s = _bcast(hsum, f)
            q0 = q0 + s * p[pl.ds(_O_W1T + 32 * f, 16)]
            q1 = q1 + s * p[pl.ds(_O_W1T + 32 * f + 16, 16)]

        bih0 = p[pl.ds(_O_BIH, 16)]
        bih1 = p[pl.ds(_O_BIH + 16, 16)]
        bhh0 = p[pl.ds(_O_BHH, 16)]
        bhh1 = p[pl.ds(_O_BHH + 16, 16)]

        def node_step(d, hm):
            hvd = hbuf[pl.ds(16 * d, 16)]
            hv8 = 8.0 * hvd
            a0, a1 = q0, q1
            for f in range(_F):
                s8 = _bcast(hv8, f)
                a0 = a0 + s8 * p[pl.ds(_O_W2T + 32 * f, 16)]
                a1 = a1 + s8 * p[pl.ds(_O_W2T + 32 * f + 16, 16)]
            gi0, gi1 = bih0, bih1
            for j in range(_H):
                s = _bcast(a0, j) if j < 16 else _bcast(a1, j - 16)
                gi0 = gi0 + s * p[pl.ds(_O_WIHT + 32 * j, 16)]
                gi1 = gi1 + s * p[pl.ds(_O_WIHT + 32 * j + 16, 16)]
            gh0, gh1 = bhh0, bhh1
            for f in range(_F):
                s = _bcast(hvd, f)
                gh0 = gh0 + s * p[pl.ds(_O_WHHT + 32 * f, 16)]
                gh1 = gh1 + s * p[pl.ds(_O_WHHT + 32 * f + 16, 16)]
            s_rz = _sigmoid(gi0 + gh0)
            n_d = _tanh(gi1 + s_rz * gh1)
            tmp[pl.ds(16, 16)] = s_rz
            zv = tmp[pl.ds(24, 16)]
            return hm + (1.0 - zv) * n_d + zv * hvd

        hm = lax.fori_loop(0, _N, node_step, zeros16)
        hm = hm * (1.0 / _N)

        decv = p[pl.ds(_O_DECW, 16)]
        prod = hm * decv
        lr_log = decb
        for f in range(_F):
            lr_log = lr_log + prod[f]
        lrv = jnp.exp(jnp.broadcast_to(lr_log, (16,)))
        lrv = jnp.minimum(jnp.maximum(lrv, 0.001), _GUARD)
        tmp[pl.ds(48, 16)] = lrv
        pltpu.sync_copy(tmp.at[pl.ds(48, 16)], out_hbm)


def _sc_call(packed):
    mesh = plsc.VectorSubcoreMesh(
        core_axis_name="c", subcore_axis_name="s", num_cores=1, num_subcores=1
    )
    return pl.kernel(
        _sc_body,
        out_type=jax.ShapeDtypeStruct((16,), jnp.float32),
        mesh=mesh,
        scratch_types=[
            pltpu.VMEM((_P_LEN,), jnp.float32),
            pltpu.VMEM((16 * _N,), jnp.float32),
            pltpu.VMEM((64,), jnp.float32),
        ],
    )(packed)


def kernel(entropy, h_param, running_time, msg_W, msg_b, gru_Wih, gru_Whh,
           gru_bih, gru_bhh, dec_W, dec_b):
    f32 = jnp.float32
    zeros8 = jnp.zeros((8,), f32)
    packed = jnp.concatenate([
        jnp.float32(entropy)[None], running_time.astype(f32),
        dec_b.astype(f32), jnp.zeros((13,), f32),
        h_param.reshape(-1).astype(f32),
        msg_W[:, :_F].T.reshape(-1).astype(f32),
        msg_W[:, _F:].T.reshape(-1).astype(f32),
        jnp.concatenate([gru_Wih.astype(f32), jnp.zeros((8, _H), f32)], axis=0).T.reshape(-1),
        jnp.concatenate([gru_Whh.astype(f32), jnp.zeros((8, _F), f32)], axis=0).T.reshape(-1),
        gru_bih.astype(f32), zeros8,
        gru_bhh.astype(f32), zeros8,
        msg_b.astype(f32),
        dec_W.reshape(-1).astype(f32), zeros8,
    ])
    out = _sc_call(packed)
    return out[:1]
